# baseline (speedup 1.0000x reference)
import jax
import jax.numpy as jnp
from jax import lax
from jax.experimental import pallas as pl
from jax.experimental.pallas import tpu as pltpu
from jax.experimental.pallas import tpu_sc as plsc

N_NODES = 50000
E_EACH = 266667
LATENT = 32

BLK_E = 2048
NBLK_E = 131
EPAD = BLK_E * NBLK_E
NIDX = EPAD // 128
IB = 4
NSUP = NIDX // IB
NPAD = 50176
DUMMY = N_NODES
BLK_N = 512
NTILES = 16


def _mlp3(x, w1, b1, w2, b2, w3, b3):
    h = jnp.maximum(jnp.dot(x, w1, preferred_element_type=jnp.float32) + b1, 0.0)
    h = jnp.maximum(jnp.dot(h, w2, preferred_element_type=jnp.float32) + b2, 0.0)
    return jnp.dot(h, w3, preferred_element_type=jnp.float32) + b3


def _mlp3_tx(xt, w1, b1, w2, b2, w3, b3):
    h = lax.dot_general(xt, w1, (((0,), (0,)), ((), ())),
                        preferred_element_type=jnp.float32)
    h = jnp.maximum(h + b1, 0.0)
    h = jnp.maximum(jnp.dot(h, w2, preferred_element_type=jnp.float32) + b2, 0.0)
    return jnp.dot(h, w3, preferred_element_type=jnp.float32) + b3


def _mlp_ln(x, w1, b1, w2, b2, w3, b3, g, be):
    h = _mlp3(x, w1, b1, w2, b2, w3, b3)
    mu = jnp.mean(h, axis=-1, keepdims=True)
    d = h - mu
    var = jnp.mean(d * d, axis=-1, keepdims=True)
    return d / jnp.sqrt(var + 1e-5) * g + be


def _mlp_ln_mxu(xt, w1, b1, w2, b2, w3, b3, g, be):
    h = _mlp3_tx(xt, w1, b1, w2, b2, w3, b3)
    avg = jnp.full((LATENT, LATENT), 1.0 / LATENT, jnp.float32)
    mu = jnp.dot(h, avg, preferred_element_type=jnp.float32)
    d = h - mu
    var = jnp.dot(d * d, avg, preferred_element_type=jnp.float32)
    return d / jnp.sqrt(var + 1e-5) * g + be


def _edge_enc_kernel(b_ref, c_ref, k_ref, w1, b1, w2, b2, w3, b3, g, be,
                     o_ref):
    for t, x_ref in enumerate((b_ref, c_ref, k_ref)):
        h = _mlp_ln_mxu(x_ref[...], w1[t], b1[t], w2[t], b2[t], w3[t],
                        b3[t], g[t], be[t])
        for c in range(BLK_E // 512):
            for q in range(4):
                base = c * 512 + q * 128
                o_ref[t, pl.ds(c * 128, 128), pl.ds(q * 32, 32)] = (
                    h[base:base + 128])


def _edge_encode(body_t, cable_t, con_t, *w):
    xspec = pl.BlockSpec((8, BLK_E), lambda i: (0, i))
    wspecs = [pl.BlockSpec(a.shape, lambda i: (0, 0, 0)) for a in w]
    return pl.pallas_call(
        _edge_enc_kernel,
        grid=(NBLK_E,),
        in_specs=[xspec, xspec, xspec] + wspecs,
        out_specs=pl.BlockSpec((3, BLK_E // 4, 128), lambda i: (0, i, 0)),
        out_shape=jax.ShapeDtypeStruct((3, EPAD // 4, 128), jnp.float32),
        compiler_params=pltpu.CompilerParams(
            dimension_semantics=("parallel",)),
    )(body_t, cable_t, con_t, *w)


def _sc_scatter_body(e_hbm, recv_hbm, zeros_hbm, out_hbm, idx_v, val_c, table,
                     ld_sem, sc_sem):
    c = lax.axis_index("c")
    s = lax.axis_index("s")
    rows = NPAD // NTILES
    zlo = s * rows
    for p in range(2):
        pltpu.sync_copy(zeros_hbm.at[pl.ds(zlo, rows)],
                        table.at[pl.ds(zlo, rows)])
        plsc.subcore_barrier()
        if p == 0:
            t = c
            sup0 = jnp.int32(0)
            nsup = NSUP
        else:
            t = jnp.int32(2)
            sup0 = c * (NSUP // 2)
            nsup = NSUP // 2
        ntrips = -(-nsup // NTILES)

        def body(i, carry):
            local = s + NTILES * i

            @pl.when(local < nsup)
            def _():
                g = sup0 + local
                loads = [pltpu.make_async_copy(
                    recv_hbm.at[t, pl.ds(g * IB, IB)], idx_v, ld_sem)]
                for j in range(IB):
                    loads.append(pltpu.make_async_copy(
                        e_hbm.at[t, pl.ds(g * 128, 128), pl.ds(j * 32, 32)],
                        val_c.at[j], ld_sem))
                for d in loads:
                    d.start()
                for d in loads:
                    d.wait()
                scats = [pltpu.make_async_copy(
                    val_c.at[j], table.at[idx_v.at[j]], sc_sem)
                    for j in range(IB)]
                for d in scats:
                    d.start(add=True)
                for d in scats:
                    d.wait()

            return carry

        lax.fori_loop(0, ntrips, body, 0)
        plsc.subcore_barrier()
        slot = 2 * p + c
        pltpu.sync_copy(table.at[pl.ds(zlo, rows)],
                        out_hbm.at[slot, pl.ds(zlo, rows)])
        plsc.subcore_barrier()


_sc_scatter = pl.kernel(
    _sc_scatter_body,
    out_type=jax.ShapeDtypeStruct((4, NPAD, 32), jnp.float32),
    mesh=plsc.VectorSubcoreMesh(core_axis_name="c", subcore_axis_name="s"),
    scratch_types=[
        pltpu.VMEM((IB, 128), jnp.int32),
        pltpu.VMEM((IB, 128, 32), jnp.float32),
        pltpu.VMEM_SHARED((NPAD, 32), jnp.float32),
        pltpu.SemaphoreType.DMA,
        pltpu.SemaphoreType.DMA,
    ],
    compiler_params=pltpu.CompilerParams(use_tc_tiling_on_sc=False),
)


def _node_enc_kernel(x_ref, w1, b1, w2, b2, w3, b3, g, be, o_ref):
    h = _mlp3_tx(x_ref[...], w1[...], b1[...], w2[...], b2[...],
                 w3[...], b3[...])
    mu = jnp.mean(h, axis=-1, keepdims=True)
    d = h - mu
    var = jnp.mean(d * d, axis=-1, keepdims=True)
    o_ref[...] = d / jnp.sqrt(var + 1e-5) * g[...] + be[...]


def _node_encode(nodes_t, *w):
    wspecs = [pl.BlockSpec(a.shape, lambda i: (0, 0)) for a in w]
    return pl.pallas_call(
        _node_enc_kernel,
        grid=(NPAD // BLK_N,),
        in_specs=[pl.BlockSpec((16, BLK_N), lambda i: (0, i))] + wspecs,
        out_specs=pl.BlockSpec((BLK_N, 32), lambda i: (i, 0)),
        out_shape=jax.ShapeDtypeStruct((N_NODES, 32), jnp.float32),
        compiler_params=pltpu.CompilerParams(
            dimension_semantics=("parallel",)),
    )(nodes_t, *w)


def _node_upd_kernel(h_ref, agg_ref,
                     uw1, ub1, uw2, ub2, uw3, ub3, ug, ube,
                     dw1, db1, dw2, db2, dw3, db3, o_ref):
    h = h_ref[...]
    a = agg_ref[...]
    w1 = uw1[...]
    base = (jnp.dot(a[0], w1[32:64], preferred_element_type=jnp.float32)
            + jnp.dot(a[1], w1[64:96], preferred_element_type=jnp.float32)
            + jnp.dot(a[2] + a[3], w1[96:128],
                      preferred_element_type=jnp.float32)
            + ub1[...])
    for _ in range(3):
        z = jnp.maximum(
            jnp.dot(h, w1[:32], preferred_element_type=jnp.float32) + base,
            0.0)
        z = jnp.maximum(
            jnp.dot(z, uw2[...], preferred_element_type=jnp.float32)
            + ub2[...], 0.0)
        z = jnp.dot(z, uw3[...], preferred_element_type=jnp.float32) + ub3[...]
        mu = jnp.mean(z, axis=-1, keepdims=True)
        d = z - mu
        var = jnp.mean(d * d, axis=-1, keepdims=True)
        h = d / jnp.sqrt(var + 1e-5) * ug[...] + ube[...]
    h = jnp.maximum(jnp.dot(h, dw1[...], preferred_element_type=jnp.float32)
                    + db1[...], 0.0)
    h = jnp.maximum(jnp.dot(h, dw2[...], preferred_element_type=jnp.float32)
                    + db2[...], 0.0)
    o_ref[...] = (jnp.dot(h, dw3[...], preferred_element_type=jnp.float32)
                  + db3[...])


def _node_update(h0, agg, *w):
    wspecs = [pl.BlockSpec(a.shape, lambda i: (0, 0)) for a in w]
    return pl.pallas_call(
        _node_upd_kernel,
        grid=(NPAD // BLK_N,),
        in_specs=[
            pl.BlockSpec((BLK_N, 32), lambda i: (i, 0)),
            pl.BlockSpec((4, BLK_N, 32), lambda i: (0, i, 0)),
        ] + wspecs,
        out_specs=pl.BlockSpec((BLK_N, 3), lambda i: (i, 0)),
        out_shape=jax.ShapeDtypeStruct((N_NODES, 3), jnp.float32),
        compiler_params=pltpu.CompilerParams(
            dimension_semantics=("parallel",)),
    )(h0, agg, *w)


def _mlp_weights(p, with_ln):
    (w1, b1), (w2, b2), (w3, b3) = p["layers"]
    ws = [w1.T, b1[None, :], w2.T, b2[None, :], w3.T, b3[None, :]]
    if with_ln:
        g, be = p["ln"]
        ws += [g[None, :], be[None, :]]
    return ws


def kernel(nodes, body, cable, con, edge_type, senders, receivers,
           p_node_enc, p_body_enc, p_cable_enc, p_con_enc,
           p_edge_upd, p_node_upd, p_dec):
    del edge_type, senders, p_edge_upd

    encs = (p_body_enc, p_cable_enc, p_con_enc)
    ew = [jnp.stack([_mlp_weights(p, True)[i] for p in encs])
          for i in range(8)]
    e = _edge_encode(body.T, cable.T, con.T, *ew)

    recv = receivers.reshape(3, E_EACH)
    recv = jnp.pad(recv, ((0, 0), (0, EPAD - E_EACH)), constant_values=DUMMY)
    recv = recv.reshape(3, NIDX, 128)
    zeros = jnp.zeros((NPAD, 32), jnp.float32)
    agg = _sc_scatter(e, recv, zeros)

    h0 = _node_encode(nodes.T, *_mlp_weights(p_node_enc, True))
    out = _node_update(h0, agg, *(_mlp_weights(p_node_upd, True)
                                  + _mlp_weights(p_dec, False)))
    return out


# speedup vs baseline: 1.0152x; 1.0152x over previous
import jax
import jax.numpy as jnp
from jax import lax
from jax.experimental import pallas as pl
from jax.experimental.pallas import tpu as pltpu
from jax.experimental.pallas import tpu_sc as plsc

N_NODES = 50000
E_EACH = 266667
LATENT = 32

BLK_E = 4096
NBLK_E = 66
EPAD = BLK_E * NBLK_E
NIDX = EPAD // 128
IB = 4
NSUP = NIDX // IB
NPAD = 50176
DUMMY = N_NODES
BLK_N = 512
NTILES = 16


def _mlp3(x, w1, b1, w2, b2, w3, b3):
    h = jnp.maximum(jnp.dot(x, w1, preferred_element_type=jnp.float32) + b1, 0.0)
    h = jnp.maximum(jnp.dot(h, w2, preferred_element_type=jnp.float32) + b2, 0.0)
    return jnp.dot(h, w3, preferred_element_type=jnp.float32) + b3


def _mlp3_tx(xt, w1, b1, w2, b2, w3, b3):
    h = lax.dot_general(xt, w1, (((0,), (0,)), ((), ())),
                        preferred_element_type=jnp.float32)
    h = jnp.maximum(h + b1, 0.0)
    h = jnp.maximum(jnp.dot(h, w2, preferred_element_type=jnp.float32) + b2, 0.0)
    return jnp.dot(h, w3, preferred_element_type=jnp.float32) + b3


def _mlp_ln(x, w1, b1, w2, b2, w3, b3, g, be):
    h = _mlp3(x, w1, b1, w2, b2, w3, b3)
    mu = jnp.mean(h, axis=-1, keepdims=True)
    d = h - mu
    var = jnp.mean(d * d, axis=-1, keepdims=True)
    return d / jnp.sqrt(var + 1e-5) * g + be


def _mlp_ln_mxu(xt, w1, b1, w2, b2, w3, b3, g, be):
    h = _mlp3_tx(xt, w1, b1, w2, b2, w3, b3)
    avg = jnp.full((LATENT, LATENT), 1.0 / LATENT, jnp.float32)
    mu = jnp.dot(h, avg, preferred_element_type=jnp.float32)
    d = h - mu
    var = jnp.dot(d * d, avg, preferred_element_type=jnp.float32)
    return d / jnp.sqrt(var + 1e-5) * g + be


def _edge_enc_kernel(b_ref, c_ref, k_ref, w1, b1, w2, b2, w3, b3, g, be,
                     o_ref):
    for t, x_ref in enumerate((b_ref, c_ref, k_ref)):
        h = _mlp_ln_mxu(x_ref[...], w1[t], b1[t], w2[t], b2[t], w3[t],
                        b3[t], g[t], be[t])
        for c in range(BLK_E // 512):
            for q in range(4):
                base = c * 512 + q * 128
                o_ref[t, pl.ds(c * 128, 128), pl.ds(q * 32, 32)] = (
                    h[base:base + 128])


def _edge_encode(body_t, cable_t, con_t, *w):
    xspec = pl.BlockSpec((8, BLK_E), lambda i: (0, i))
    wspecs = [pl.BlockSpec(a.shape, lambda i: (0, 0, 0)) for a in w]
    return pl.pallas_call(
        _edge_enc_kernel,
        grid=(NBLK_E,),
        in_specs=[xspec, xspec, xspec] + wspecs,
        out_specs=pl.BlockSpec((3, BLK_E // 4, 128), lambda i: (0, i, 0)),
        out_shape=jax.ShapeDtypeStruct((3, EPAD // 4, 128), jnp.float32),
        compiler_params=pltpu.CompilerParams(
            dimension_semantics=("parallel",)),
    )(body_t, cable_t, con_t, *w)


def _sc_scatter_body(e_hbm, recv_hbm, zeros_hbm, out_hbm, idx_v, val_c, table,
                     ld_sem, sc_sem):
    c = lax.axis_index("c")
    s = lax.axis_index("s")
    rows = NPAD // NTILES
    zlo = s * rows
    for p in range(2):
        pltpu.sync_copy(zeros_hbm.at[pl.ds(zlo, rows)],
                        table.at[pl.ds(zlo, rows)])
        plsc.subcore_barrier()
        if p == 0:
            t = c
            sup0 = jnp.int32(0)
            nsup = NSUP
        else:
            t = jnp.int32(2)
            sup0 = c * (NSUP // 2)
            nsup = NSUP // 2
        ntrips = -(-nsup // NTILES)

        def body(i, carry):
            local = s + NTILES * i

            @pl.when(local < nsup)
            def _():
                g = sup0 + local
                loads = [pltpu.make_async_copy(
                    recv_hbm.at[t, pl.ds(g * IB, IB)], idx_v, ld_sem)]
                for j in range(IB):
                    loads.append(pltpu.make_async_copy(
                        e_hbm.at[t, pl.ds(g * 128, 128), pl.ds(j * 32, 32)],
                        val_c.at[j], ld_sem))
                for d in loads:
                    d.start()
                for d in loads:
                    d.wait()
                scats = [pltpu.make_async_copy(
                    val_c.at[j], table.at[idx_v.at[j]], sc_sem)
                    for j in range(IB)]
                for d in scats:
                    d.start(add=True)
                for d in scats:
                    d.wait()

            return carry

        lax.fori_loop(0, ntrips, body, 0)
        plsc.subcore_barrier()
        slot = 2 * p + c
        pltpu.sync_copy(table.at[pl.ds(zlo, rows)],
                        out_hbm.at[slot, pl.ds(zlo, rows)])
        plsc.subcore_barrier()


_sc_scatter = pl.kernel(
    _sc_scatter_body,
    out_type=jax.ShapeDtypeStruct((4, NPAD, 32), jnp.float32),
    mesh=plsc.VectorSubcoreMesh(core_axis_name="c", subcore_axis_name="s"),
    scratch_types=[
        pltpu.VMEM((IB, 128), jnp.int32),
        pltpu.VMEM((IB, 128, 32), jnp.float32),
        pltpu.VMEM_SHARED((NPAD, 32), jnp.float32),
        pltpu.SemaphoreType.DMA,
        pltpu.SemaphoreType.DMA,
    ],
    compiler_params=pltpu.CompilerParams(use_tc_tiling_on_sc=False),
)


def _node_enc_kernel(x_ref, w1, b1, w2, b2, w3, b3, g, be, o_ref):
    h = _mlp3_tx(x_ref[...], w1[...], b1[...], w2[...], b2[...],
                 w3[...], b3[...])
    mu = jnp.mean(h, axis=-1, keepdims=True)
    d = h - mu
    var = jnp.mean(d * d, axis=-1, keepdims=True)
    o_ref[...] = d / jnp.sqrt(var + 1e-5) * g[...] + be[...]


def _node_encode(nodes_t, *w):
    wspecs = [pl.BlockSpec(a.shape, lambda i: (0, 0)) for a in w]
    return pl.pallas_call(
        _node_enc_kernel,
        grid=(NPAD // BLK_N,),
        in_specs=[pl.BlockSpec((16, BLK_N), lambda i: (0, i))] + wspecs,
        out_specs=pl.BlockSpec((BLK_N, 32), lambda i: (i, 0)),
        out_shape=jax.ShapeDtypeStruct((N_NODES, 32), jnp.float32),
        compiler_params=pltpu.CompilerParams(
            dimension_semantics=("parallel",)),
    )(nodes_t, *w)


def _node_upd_kernel(h_ref, agg_ref,
                     uw1, ub1, uw2, ub2, uw3, ub3, ug, ube,
                     dw1, db1, dw2, db2, dw3, db3, o_ref):
    h = h_ref[...]
    a = agg_ref[...]
    agg = jnp.concatenate([a[0], a[1], a[2] + a[3]], axis=-1)
    for _ in range(3):
        h = _mlp_ln(jnp.concatenate([h, agg], axis=-1),
                    uw1[...], ub1[...], uw2[...], ub2[...],
                    uw3[...], ub3[...], ug[...], ube[...])
    h = jnp.maximum(jnp.dot(h, dw1[...], preferred_element_type=jnp.float32)
                    + db1[...], 0.0)
    h = jnp.maximum(jnp.dot(h, dw2[...], preferred_element_type=jnp.float32)
                    + db2[...], 0.0)
    o_ref[...] = (jnp.dot(h, dw3[...], preferred_element_type=jnp.float32)
                  + db3[...])


def _node_update(h0, agg, *w):
    wspecs = [pl.BlockSpec(a.shape, lambda i: (0, 0)) for a in w]
    return pl.pallas_call(
        _node_upd_kernel,
        grid=(NPAD // BLK_N,),
        in_specs=[
            pl.BlockSpec((BLK_N, 32), lambda i: (i, 0)),
            pl.BlockSpec((4, BLK_N, 32), lambda i: (0, i, 0)),
        ] + wspecs,
        out_specs=pl.BlockSpec((BLK_N, 3), lambda i: (i, 0)),
        out_shape=jax.ShapeDtypeStruct((N_NODES, 3), jnp.float32),
        compiler_params=pltpu.CompilerParams(
            dimension_semantics=("parallel",)),
    )(h0, agg, *w)


def _mlp_weights(p, with_ln):
    (w1, b1), (w2, b2), (w3, b3) = p["layers"]
    ws = [w1.T, b1[None, :], w2.T, b2[None, :], w3.T, b3[None, :]]
    if with_ln:
        g, be = p["ln"]
        ws += [g[None, :], be[None, :]]
    return ws


def kernel(nodes, body, cable, con, edge_type, senders, receivers,
           p_node_enc, p_body_enc, p_cable_enc, p_con_enc,
           p_edge_upd, p_node_upd, p_dec):
    del edge_type, senders, p_edge_upd

    encs = (p_body_enc, p_cable_enc, p_con_enc)
    ew = [jnp.stack([_mlp_weights(p, True)[i] for p in encs])
          for i in range(8)]
    e = _edge_encode(body.T, cable.T, con.T, *ew)

    recv = receivers.reshape(3, E_EACH)
    recv = jnp.pad(recv, ((0, 0), (0, EPAD - E_EACH)), constant_values=DUMMY)
    recv = recv.reshape(3, NIDX, 128)
    zeros = jnp.zeros((NPAD, 32), jnp.float32)
    agg = _sc_scatter(e, recv, zeros)

    h0 = _node_encode(nodes.T, *_mlp_weights(p_node_enc, True))
    out = _node_update(h0, agg, *(_mlp_weights(p_node_upd, True)
                                  + _mlp_weights(p_dec, False)))
    return out


# speedup vs baseline: 1.1297x; 1.1127x over previous
import jax
import jax.numpy as jnp
from jax import lax
from jax.experimental import pallas as pl
from jax.experimental.pallas import tpu as pltpu
from jax.experimental.pallas import tpu_sc as plsc

N_NODES = 50000
E_EACH = 266667
LATENT = 32

BLK_E = 4096
NBLK_E = 66
EPAD = BLK_E * NBLK_E
NIDX = EPAD // 128
IB = 4
NSUP = NIDX // IB
NPAD = 50176
DUMMY = N_NODES
BLK_N = 1024
NTILES = 16


def _mlp3(x, w1, b1, w2, b2, w3, b3):
    h = jnp.maximum(jnp.dot(x, w1, preferred_element_type=jnp.float32) + b1, 0.0)
    h = jnp.maximum(jnp.dot(h, w2, preferred_element_type=jnp.float32) + b2, 0.0)
    return jnp.dot(h, w3, preferred_element_type=jnp.float32) + b3


def _mlp3_tx(xt, w1, b1, w2, b2, w3, b3):
    h = lax.dot_general(xt, w1, (((0,), (0,)), ((), ())),
                        preferred_element_type=jnp.float32)
    h = jnp.maximum(h + b1, 0.0)
    h = jnp.maximum(jnp.dot(h, w2, preferred_element_type=jnp.float32) + b2, 0.0)
    return jnp.dot(h, w3, preferred_element_type=jnp.float32) + b3


def _mlp_ln(x, w1, b1, w2, b2, w3, b3, g, be):
    h = _mlp3(x, w1, b1, w2, b2, w3, b3)
    mu = jnp.mean(h, axis=-1, keepdims=True)
    d = h - mu
    var = jnp.mean(d * d, axis=-1, keepdims=True)
    return d / jnp.sqrt(var + 1e-5) * g + be


def _mlp_ln_mxu(xt, w1, b1, w2, b2, w3, b3, g, be):
    h = _mlp3_tx(xt, w1, b1, w2, b2, w3, b3)
    avg = jnp.full((LATENT, LATENT), 1.0 / LATENT, jnp.float32)
    mu = jnp.dot(h, avg, preferred_element_type=jnp.float32)
    d = h - mu
    var = jnp.dot(d * d, avg, preferred_element_type=jnp.float32)
    return d / jnp.sqrt(var + 1e-5) * g + be


def _edge_enc_kernel(b_ref, c_ref, k_ref, w1, b1, w2, b2, w3, b3, g, be,
                     o_ref):
    for t, x_ref in enumerate((b_ref, c_ref, k_ref)):
        h = _mlp_ln_mxu(x_ref[...], w1[t], b1[t], w2[t], b2[t], w3[t],
                        b3[t], g[t], be[t])
        for c in range(BLK_E // 512):
            for q in range(4):
                base = c * 512 + q * 128
                o_ref[t, pl.ds(c * 128, 128), pl.ds(q * 32, 32)] = (
                    h[base:base + 128])


def _edge_encode(body_t, cable_t, con_t, *w):
    xspec = pl.BlockSpec((8, BLK_E), lambda i: (0, i))
    wspecs = [pl.BlockSpec(a.shape, lambda i: (0, 0, 0)) for a in w]
    return pl.pallas_call(
        _edge_enc_kernel,
        grid=(NBLK_E,),
        in_specs=[xspec, xspec, xspec] + wspecs,
        out_specs=pl.BlockSpec((3, BLK_E // 4, 128), lambda i: (0, i, 0)),
        out_shape=jax.ShapeDtypeStruct((3, EPAD // 4, 128), jnp.float32),
        compiler_params=pltpu.CompilerParams(
            dimension_semantics=("parallel",)),
    )(body_t, cable_t, con_t, *w)


def _sc_scatter_body(e_hbm, recv_hbm, zeros_hbm, out_hbm, idx_v, val_c, table,
                     ld_sem, sc_sem):
    c = lax.axis_index("c")
    s = lax.axis_index("s")
    rows = NPAD // NTILES
    zlo = s * rows
    for p in range(2):
        pltpu.sync_copy(zeros_hbm.at[pl.ds(zlo, rows)],
                        table.at[pl.ds(zlo, rows)])
        plsc.subcore_barrier()
        if p == 0:
            t = c
            sup0 = jnp.int32(0)
            nsup = NSUP
        else:
            t = jnp.int32(2)
            sup0 = c * (NSUP // 2)
            nsup = NSUP // 2
        ntrips = -(-nsup // NTILES)

        def body(i, carry):
            local = s + NTILES * i

            @pl.when(local < nsup)
            def _():
                g = sup0 + local
                loads = [pltpu.make_async_copy(
                    recv_hbm.at[t, pl.ds(g * IB, IB)], idx_v, ld_sem)]
                for j in range(IB):
                    loads.append(pltpu.make_async_copy(
                        e_hbm.at[t, pl.ds(g * 128, 128), pl.ds(j * 32, 32)],
                        val_c.at[j], ld_sem))
                for d in loads:
                    d.start()
                for d in loads:
                    d.wait()
                scats = [pltpu.make_async_copy(
                    val_c.at[j], table.at[idx_v.at[j]], sc_sem)
                    for j in range(IB)]
                for d in scats:
                    d.start(add=True)
                for d in scats:
                    d.wait()

            return carry

        lax.fori_loop(0, ntrips, body, 0)
        plsc.subcore_barrier()
        slot = 2 * p + c
        pltpu.sync_copy(table.at[pl.ds(zlo, rows)],
                        out_hbm.at[slot, pl.ds(zlo, rows)])
        plsc.subcore_barrier()


_sc_scatter = pl.kernel(
    _sc_scatter_body,
    out_type=jax.ShapeDtypeStruct((4, NPAD, 32), jnp.float32),
    mesh=plsc.VectorSubcoreMesh(core_axis_name="c", subcore_axis_name="s"),
    scratch_types=[
        pltpu.VMEM((IB, 128), jnp.int32),
        pltpu.VMEM((IB, 128, 32), jnp.float32),
        pltpu.VMEM_SHARED((NPAD, 32), jnp.float32),
        pltpu.SemaphoreType.DMA,
        pltpu.SemaphoreType.DMA,
    ],
    compiler_params=pltpu.CompilerParams(use_tc_tiling_on_sc=False),
)


def _node_enc_kernel(x_ref, w1, b1, w2, b2, w3, b3, g, be, o_ref):
    h = _mlp3_tx(x_ref[...], w1[...], b1[...], w2[...], b2[...],
                 w3[...], b3[...])
    mu = jnp.mean(h, axis=-1, keepdims=True)
    d = h - mu
    var = jnp.mean(d * d, axis=-1, keepdims=True)
    o_ref[...] = d / jnp.sqrt(var + 1e-5) * g[...] + be[...]


def _node_encode(nodes_t, *w):
    wspecs = [pl.BlockSpec(a.shape, lambda i: (0, 0)) for a in w]
    return pl.pallas_call(
        _node_enc_kernel,
        grid=(NPAD // BLK_N,),
        in_specs=[pl.BlockSpec((16, BLK_N), lambda i: (0, i))] + wspecs,
        out_specs=pl.BlockSpec((BLK_N, 32), lambda i: (i, 0)),
        out_shape=jax.ShapeDtypeStruct((N_NODES, 32), jnp.float32),
        compiler_params=pltpu.CompilerParams(
            dimension_semantics=("parallel",)),
    )(nodes_t, *w)


def _node_upd_kernel(h_ref, agg_ref,
                     uw1, ub1, uw2, ub2, uw3, ub3, ug, ube,
                     dw1, db1, dw2, db2, dw3, db3, o_ref):
    h = h_ref[...]
    a = agg_ref[...]
    agg = jnp.concatenate([a[0], a[1], a[2] + a[3]], axis=-1)
    for _ in range(3):
        h = _mlp_ln(jnp.concatenate([h, agg], axis=-1),
                    uw1[...], ub1[...], uw2[...], ub2[...],
                    uw3[...], ub3[...], ug[...], ube[...])
    h = jnp.maximum(jnp.dot(h, dw1[...], preferred_element_type=jnp.float32)
                    + db1[...], 0.0)
    h = jnp.maximum(jnp.dot(h, dw2[...], preferred_element_type=jnp.float32)
                    + db2[...], 0.0)
    o_ref[...] = (jnp.dot(h, dw3[...], preferred_element_type=jnp.float32)
                  + db3[...])


def _node_update(h0, agg, *w):
    wspecs = [pl.BlockSpec(a.shape, lambda i: (0, 0)) for a in w]
    return pl.pallas_call(
        _node_upd_kernel,
        grid=(NPAD // BLK_N,),
        in_specs=[
            pl.BlockSpec((BLK_N, 32), lambda i: (i, 0)),
            pl.BlockSpec((4, BLK_N, 32), lambda i: (0, i, 0)),
        ] + wspecs,
        out_specs=pl.BlockSpec((BLK_N, 3), lambda i: (i, 0)),
        out_shape=jax.ShapeDtypeStruct((N_NODES, 3), jnp.float32),
        compiler_params=pltpu.CompilerParams(
            dimension_semantics=("parallel",)),
    )(h0, agg, *w)


def _mlp_weights(p, with_ln):
    (w1, b1), (w2, b2), (w3, b3) = p["layers"]
    ws = [w1.T, b1[None, :], w2.T, b2[None, :], w3.T, b3[None, :]]
    if with_ln:
        g, be = p["ln"]
        ws += [g[None, :], be[None, :]]
    return ws


def kernel(nodes, body, cable, con, edge_type, senders, receivers,
           p_node_enc, p_body_enc, p_cable_enc, p_con_enc,
           p_edge_upd, p_node_upd, p_dec):
    del edge_type, senders, p_edge_upd

    encs = (p_body_enc, p_cable_enc, p_con_enc)
    ew = [jnp.stack([_mlp_weights(p, True)[i] for p in encs])
          for i in range(8)]
    e = _edge_encode(body.T, cable.T, con.T, *ew)

    recv = receivers.reshape(3, E_EACH)
    recv = jnp.pad(recv, ((0, 0), (0, EPAD - E_EACH)), constant_values=DUMMY)
    recv = recv.reshape(3, NIDX, 128)
    zeros = jnp.zeros((NPAD, 32), jnp.float32)
    agg = _sc_scatter(e, recv, zeros)

    h0 = _node_encode(nodes.T, *_mlp_weights(p_node_enc, True))
    out = _node_update(h0, agg, *(_mlp_weights(p_node_upd, True)
                                  + _mlp_weights(p_dec, False)))
    return out
